# baseline (speedup 1.0000x reference)
import jax
import jax.numpy as jnp
from jax.experimental import pallas as pl
from jax.experimental.pallas import tpu as pltpu

_N, _D = 1000000, 64
_W = 512
_RPR = _W // _D
_NR = _N * _D // _W
_BR = 1000
_GRID = _NR // _BR


def _body(idx_ref, val_ref, x_ref, o_ref):
    i = pl.program_id(0)
    idx0 = idx_ref[0]
    idx1 = idx_ref[1]
    lo = i * (_BR * _RPR)
    hi = lo + _BR * _RPR
    hit0 = jnp.logical_and(idx0 >= lo, idx0 < hi)
    hit1 = jnp.logical_and(idx1 >= lo, idx1 < hi)

    @pl.when(jnp.logical_or(hit0, hit1))
    def _():
        r = jax.lax.broadcasted_iota(jnp.int32, (_BR, _W), 0)
        c = jax.lax.broadcasted_iota(jnp.int32, (_BR, _W), 1)
        rows = lo + r * _RPR + c // _D
        cnt = (rows == idx0).astype(jnp.float32) + (rows == idx1).astype(
            jnp.float32)
        o_ref[...] = x_ref[...] + val_ref[0] * cnt

    @pl.when(jnp.logical_not(jnp.logical_or(hit0, hit1)))
    def _():
        o_ref[...] = x_ref[...]


def kernel(x, indices, values):
    x2 = x.reshape(_NR, _W)
    out = pl.pallas_call(
        _body,
        grid=(_GRID,),
        in_specs=[
            pl.BlockSpec(memory_space=pltpu.SMEM),
            pl.BlockSpec(memory_space=pltpu.SMEM),
            pl.BlockSpec((_BR, _W), lambda i: (i, 0)),
        ],
        out_specs=pl.BlockSpec((_BR, _W), lambda i: (i, 0)),
        out_shape=jax.ShapeDtypeStruct((_NR, _W), jnp.float32),
    )(indices, values.reshape(1), x2)
    return out.reshape(_N, _D)


# speedup vs baseline: 1.3947x; 1.3947x over previous
import jax
import jax.numpy as jnp
from jax.experimental import pallas as pl
from jax.experimental.pallas import tpu as pltpu

_N, _D = 1000000, 64
_BR = 8000
_GRID = _N // _BR


def _body(idx_ref, val_ref, x_ref, o_ref):
    i = pl.program_id(0)
    idx0 = idx_ref[0]
    idx1 = idx_ref[1]
    lo = i * _BR
    hi = lo + _BR
    hit0 = jnp.logical_and(idx0 >= lo, idx0 < hi)
    hit1 = jnp.logical_and(idx1 >= lo, idx1 < hi)

    @pl.when(jnp.logical_or(hit0, hit1))
    def _():
        rows = lo + jax.lax.broadcasted_iota(jnp.int32, (_BR, _D), 0)
        cnt = (rows == idx0).astype(jnp.float32) + (rows == idx1).astype(
            jnp.float32)
        o_ref[...] = x_ref[...] + val_ref[0] * cnt

    @pl.when(jnp.logical_not(jnp.logical_or(hit0, hit1)))
    def _():
        o_ref[...] = x_ref[...]


def kernel(x, indices, values):
    return pl.pallas_call(
        _body,
        grid=(_GRID,),
        in_specs=[
            pl.BlockSpec(memory_space=pltpu.SMEM),
            pl.BlockSpec(memory_space=pltpu.SMEM),
            pl.BlockSpec((_BR, _D), lambda i: (i, 0)),
        ],
        out_specs=pl.BlockSpec((_BR, _D), lambda i: (i, 0)),
        out_shape=jax.ShapeDtypeStruct((_N, _D), jnp.float32),
    )(indices, values.reshape(1), x)
